# baseline (speedup 1.0000x reference)
import jax
import jax.numpy as jnp
from jax.experimental import pallas as pl
from jax.experimental.pallas import tpu as pltpu


def _affine_kernel(x_ref, w_ref, b_ref, o_ref):
    g = pl.program_id(1)
    tm = x_ref.shape[0]
    xb = x_ref[...].astype(jnp.bfloat16)
    acc = jnp.dot(xb, w_ref[...], preferred_element_type=jnp.float32)
    o_ref[pl.ds(g * tm, tm), :] = acc + b_ref[...]


def kernel(x, weight, bias):
    B, K = x.shape
    N = weight.shape[0]
    w_t = weight.T.astype(jnp.bfloat16)
    b2 = bias.reshape(1, N)

    n_cores = 2 if B % 2048 == 0 else 1
    half = B // n_cores
    tm = 512
    while half % tm != 0:
        tm //= 2
    inner = half // tm

    cost = pl.CostEstimate(
        flops=2 * B * K * N,
        transcendentals=0,
        bytes_accessed=4 * B * K + 2 * K * N + 4 * B * N,
    )

    return pl.pallas_call(
        _affine_kernel,
        out_shape=jax.ShapeDtypeStruct((B, N), x.dtype),
        grid=(n_cores, inner),
        in_specs=[
            pl.BlockSpec((tm, K), lambda c, g, n=inner: (c * n + g, 0)),
            pl.BlockSpec((K, N), lambda c, g: (0, 0)),
            pl.BlockSpec((1, N), lambda c, g: (0, 0)),
        ],
        out_specs=pl.BlockSpec((half, N), lambda c, g: (c, 0)),
        compiler_params=pltpu.CompilerParams(
            dimension_semantics=("parallel", "arbitrary"),
            vmem_limit_bytes=60000 * 1024,
        ),
        cost_estimate=cost,
    )(x, w_t, b2)
